# baseline (speedup 1.0000x reference)
import jax
import jax.numpy as jnp
from jax import lax
from jax.experimental import pallas as pl
from jax.experimental.pallas import tpu as pltpu

G = 8


def _routing_dispatch_body(x_ref, rw_ref, ei_ref, cs1_ref, g1_ref, cs2_ref,
                           g2_ref):
    _, S, MM = x_ref.shape
    E = rw_ref.shape[1]
    C = S // E
    P = E * C

    x = x_ref[0]
    logits = jnp.dot(x, rw_ref[...])
    m = jnp.max(logits, axis=-1, keepdims=True)
    ex = jnp.exp(logits - m)
    raw = ex / jnp.sum(ex, axis=-1, keepdims=True)

    e_iota = lax.broadcasted_iota(jnp.int32, (S, E), 1)

    gate1 = jnp.max(raw, axis=-1)
    idx1 = jnp.min(jnp.where(raw == gate1[:, None], e_iota, E), axis=-1)
    mask1 = (e_iota == idx1[:, None]).astype(jnp.float32)

    raw2 = raw * (1.0 - mask1)
    gate2 = jnp.max(raw2, axis=-1)
    idx2 = jnp.min(jnp.where(raw2 == gate2[:, None], e_iota, E), axis=-1)
    mask2 = (e_iota == idx2[:, None]).astype(jnp.float32)

    r_iota = lax.broadcasted_iota(jnp.int32, (S, S), 0)
    c_iota = lax.broadcasted_iota(jnp.int32, (S, S), 1)
    tril = (r_iota > c_iota).astype(jnp.float32)
    pos1_all = jnp.dot(tril, mask1)
    keep1 = (pos1_all < C) & (mask1 > 0.0)
    mask1c = jnp.where(keep1, 1.0, 0.0)
    pos1 = jnp.sum(pos1_all * mask1c, axis=-1)
    kept1 = jnp.sum(mask1c, axis=-1)
    count1 = jnp.sum(mask1c, axis=0)

    pos2_all = jnp.dot(tril, mask2) + count1[None, :]
    keep2 = (pos2_all < C) & (mask2 > 0.0)
    mask2c = jnp.where(keep2, 1.0, 0.0)
    pos2 = jnp.sum(pos2_all * mask2c, axis=-1)
    kept2 = jnp.sum(mask2c, axis=-1)

    k1 = kept1 > 0.0
    k2 = kept2 > 0.0
    slot1 = jnp.where(k1, idx1 * C + pos1.astype(jnp.int32), -1)
    slot2 = jnp.where(k2, idx2 * C + pos2.astype(jnp.int32), -1)

    p_iota = lax.broadcasted_iota(jnp.int32, (S, P), 1)
    disp = ((p_iota == slot1[:, None]) | (p_iota == slot2[:, None]))
    disp = disp.astype(jnp.float32)
    ei_ref[0] = lax.dot_general(
        disp, x, dimension_numbers=(((0,), (0,)), ((), ())))

    cs1_ref[...] = slot1[None, None, :]
    cs2_ref[...] = slot2[None, None, :]
    g1_ref[...] = (gate1 * kept1)[None, None, :]
    g2_ref[...] = (gate2 * kept2)[None, None, :]


def _routing_dispatch(x, router_w, interpret=False):
    G_, S, M = x.shape
    E = router_w.shape[1]
    P = S
    i32 = jax.ShapeDtypeStruct((G_, 1, S), jnp.int32)
    f32 = jax.ShapeDtypeStruct((G_, 1, S), jnp.float32)
    ei = jax.ShapeDtypeStruct((G_, P, M), jnp.float32)
    sl_spec = pl.BlockSpec((1, 1, S), lambda g: (g, 0, 0))
    return pl.pallas_call(
        _routing_dispatch_body,
        grid=(G_,),
        in_specs=[
            pl.BlockSpec((1, S, M), lambda g: (g, 0, 0)),
            pl.BlockSpec((M, E), lambda g: (0, 0)),
        ],
        out_specs=[pl.BlockSpec((1, P, M), lambda g: (g, 0, 0)),
                   sl_spec, sl_spec, sl_spec, sl_spec],
        out_shape=[ei, i32, f32, i32, f32],
        interpret=interpret,
    )(x, router_w)


def _ffn(ei, w0, w1, wo, *, hc=2048, interpret=False):
    E, M, H = w0.shape
    G_, P, _ = ei.shape
    C = P // E
    grid = (E, H // hc)

    def body(ei_ref, w0_ref, w1_ref, wo_ref, out_ref):
        h = pl.program_id(1)
        a = ei_ref[...].reshape(G_ * C, M)
        h0 = jnp.dot(a, w0_ref[0])
        h1 = jnp.dot(a, w1_ref[0])
        part = jnp.dot(jax.nn.gelu(h0) * h1, wo_ref[0])

        @pl.when(h == 0)
        def _():
            out_ref[...] = part.reshape(1, G_, C, M)

        @pl.when(h > 0)
        def _():
            out_ref[...] += part.reshape(1, G_, C, M)

    return pl.pallas_call(
        body,
        grid=grid,
        in_specs=[
            pl.BlockSpec((G_, C, M), lambda e, h: (0, e, 0)),
            pl.BlockSpec((1, M, hc), lambda e, h: (e, 0, h)),
            pl.BlockSpec((1, M, hc), lambda e, h: (e, 0, h)),
            pl.BlockSpec((1, hc, M), lambda e, h: (e, h, 0)),
        ],
        out_specs=pl.BlockSpec((1, G_, C, M), lambda e, h: (e, 0, 0, 0)),
        out_shape=jax.ShapeDtypeStruct((E, G_, C, M), jnp.float32),
        compiler_params=pltpu.CompilerParams(
            dimension_semantics=("parallel", "arbitrary"),
        ),
        interpret=interpret,
    )(ei, w0, w1, wo)


def _combine(eo, cs1, g1, cs2, g2, interpret=False):
    E, G_, C, M = eo.shape
    S = cs1.shape[2]
    P = E * C

    def body(eo_ref, cs1_ref, g1_ref, cs2_ref, g2_ref, out_ref):
        eo_mat = eo_ref[...].reshape(P, M)
        s1 = cs1_ref[0, 0, :]
        s2 = cs2_ref[0, 0, :]
        ga = g1_ref[0, 0, :]
        gb = g2_ref[0, 0, :]
        p_iota = lax.broadcasted_iota(jnp.int32, (S, P), 1)
        cmb = (jnp.where(p_iota == s1[:, None], ga[:, None], 0.0)
               + jnp.where(p_iota == s2[:, None], gb[:, None], 0.0))
        out_ref[0] = jnp.dot(cmb, eo_mat)

    sl_spec = pl.BlockSpec((1, 1, S), lambda g: (g, 0, 0))
    return pl.pallas_call(
        body,
        grid=(G_,),
        in_specs=[
            pl.BlockSpec((E, 1, C, M), lambda g: (0, g, 0, 0)),
            sl_spec, sl_spec, sl_spec, sl_spec,
        ],
        out_specs=pl.BlockSpec((1, S, M), lambda g: (g, 0, 0)),
        out_shape=jax.ShapeDtypeStruct((G_, S, M), jnp.float32),
        interpret=interpret,
    )(eo, cs1, g1, cs2, g2)


def kernel(inputs, router_w, w0, w1, wo):
    B, L, M = inputs.shape
    S = B * L // G
    x = inputs.reshape(G, S, M)

    ei, cs1, g1, cs2, g2 = _routing_dispatch(x, router_w)
    eo = _ffn(ei, w0, w1, wo)
    out = _combine(eo, cs1, g1, cs2, g2)
    return out.reshape(B, L, M)
